# baseline (speedup 1.0000x reference)
import functools

import jax
import jax.numpy as jnp
from jax import lax
from jax.experimental import pallas as pl
from jax.experimental.pallas import tpu as pltpu
from jax.experimental.pallas import tpu_sc as plsc

N = 10000
D = 128
E = 320000
NL = 65536
NC, NS = 2, 16
NW = NC * NS
CH = 128
NCHUNK = 80
E_PAD = NW * NCHUNK * CH
NSTG = 2
CPS = NCHUNK // NSTG
PPS = CPS // 2
ACC_PAD = 10240
RPT = ACC_PAD // NS
ZR = 128
CNT_PAD = 10240
CPT = CNT_PAD // NS
LPW = NL // NW

_mesh = plsc.VectorSubcoreMesh(core_axis_name="c", subcore_axis_name="s")


def _zero_f32(ref, nwords):
    def body(i, _):
        ref[pl.ds(i * 16, 16)] = jnp.zeros((16,), jnp.float32)
        return 0
    lax.fori_loop(0, nwords // 16, body, 0)


def _agg_body(with_counts, x_hbm, src_hbm, dst_hbm, acc_out, cnt_out,
              idx_s, idx_d, rows_a, rows_b, ones_v, zcnt_v,
              acc_sh, cnt_sh, gsa, gsb, ssa, ssb, csa, csb):
    cid = lax.axis_index("c")
    sid = lax.axis_index("s")
    gwid = cid * NS + sid

    def zrow_fill(i, _):
        rows_a[i // (D // 16), pl.ds((i % (D // 16)) * 16, 16)] = (
            jnp.zeros((16,), jnp.float32))
        return 0
    lax.fori_loop(0, ZR * (D // 16), zrow_fill, 0)

    def zacc(j, _):
        pltpu.sync_copy(rows_a, acc_sh.at[pl.ds(sid * RPT + j * ZR, ZR)])
        return 0
    lax.fori_loop(0, RPT // ZR, zacc, 0)

    if with_counts:
        def ones_fill(i, _):
            ones_v[pl.ds(i * 16, 16)] = jnp.ones((16,), jnp.float32)
            return 0
        lax.fori_loop(0, CH // 16, ones_fill, 0)
        _zero_f32(zcnt_v, CPT)
        pltpu.sync_copy(zcnt_v, cnt_sh.at[pl.ds(sid * CPT, CPT)])

    plsc.subcore_barrier()

    for st in range(NSTG):
        pltpu.sync_copy(src_hbm.at[gwid, pl.ds(st * CPS, CPS)], idx_s)
        pltpu.sync_copy(dst_hbm.at[gwid, pl.ds(st * CPS, CPS)], idx_d)
        pltpu.async_copy(x_hbm.at[idx_s.at[0]], rows_a, gsa)

        def pair(k2, _):
            ka = 2 * k2
            kb = ka + 1
            gb = pltpu.async_copy(x_hbm.at[idx_s.at[kb]], rows_b, gsb)
            pltpu.make_async_copy(x_hbm.at[idx_s.at[ka]], rows_a, gsa).wait()
            sa = pltpu.async_copy(rows_a, acc_sh.at[idx_d.at[ka]], ssa,
                                  add=True)
            if with_counts:
                ca = pltpu.async_copy(ones_v, cnt_sh.at[idx_d.at[ka]], csa,
                                      add=True)
            gb.wait()
            sb = pltpu.async_copy(rows_b, acc_sh.at[idx_d.at[kb]], ssb,
                                  add=True)
            if with_counts:
                cb = pltpu.async_copy(ones_v, cnt_sh.at[idx_d.at[kb]], csb,
                                      add=True)
            sa.wait()
            if with_counts:
                ca.wait()

            @pl.when(k2 < PPS - 1)
            def _():
                pltpu.async_copy(x_hbm.at[idx_s.at[ka + 2]], rows_a, gsa)
            sb.wait()
            if with_counts:
                cb.wait()
            return 0
        lax.fori_loop(0, PPS, pair, 0)

    plsc.subcore_barrier()

    def cout(j, _):
        r = pl.multiple_of(sid * RPT + j * ZR, 8)
        pltpu.sync_copy(acc_sh.at[pl.ds(r, ZR)], acc_out.at[cid, pl.ds(r, ZR)])
        return 0
    lax.fori_loop(0, RPT // ZR, cout, 0)
    if with_counts:
        pltpu.sync_copy(cnt_sh.at[pl.ds(sid * CPT, CPT)],
                        cnt_out.at[cid, pl.ds(sid * CPT, CPT)])


def _make_agg(with_counts):
    out_type = (jax.ShapeDtypeStruct((NC, ACC_PAD, D), jnp.float32),)
    if with_counts:
        out_type = out_type + (jax.ShapeDtypeStruct((NC, CNT_PAD), jnp.float32),)
    scratch = [
        pltpu.VMEM((CPS, CH), jnp.int32),
        pltpu.VMEM((CPS, CH), jnp.int32),
        pltpu.VMEM((CH, D), jnp.float32),
        pltpu.VMEM((CH, D), jnp.float32),
        pltpu.VMEM((CH,), jnp.float32),
        pltpu.VMEM((CPT,), jnp.float32),
        pltpu.VMEM_SHARED((ACC_PAD, D), jnp.float32),
        pltpu.VMEM_SHARED((CNT_PAD,), jnp.float32),
        pltpu.SemaphoreType.DMA,
        pltpu.SemaphoreType.DMA,
        pltpu.SemaphoreType.DMA,
        pltpu.SemaphoreType.DMA,
        pltpu.SemaphoreType.DMA,
        pltpu.SemaphoreType.DMA,
    ]
    body = functools.partial(_agg_body, with_counts)
    if not with_counts:
        def body(x_hbm, src_hbm, dst_hbm, acc_out, *rest):
            return _agg_body(False, x_hbm, src_hbm, dst_hbm, acc_out, None, *rest)
    return pl.kernel(body, out_type=out_type, mesh=_mesh, scratch_types=scratch)


_agg_with_counts = _make_agg(True)
_agg_no_counts = _make_agg(False)


def _layer1_tc(acc_ref, cnt0_ref, cnt1_ref, x_ref, wl_ref, wr_ref, bl_ref, o_ref):
    cnt = jnp.maximum(cnt0_ref[...] + cnt1_ref[...], 1.0)
    mean = (acc_ref[0] + acc_ref[1]) / cnt
    h = (lax.dot_general(mean, wl_ref[...], (((1,), (1,)), ((), ())),
                         preferred_element_type=jnp.float32)
         + lax.dot_general(x_ref[...], wr_ref[...], (((1,), (1,)), ((), ())),
                           preferred_element_type=jnp.float32)
         + bl_ref[...])
    o_ref[...] = jnp.maximum(h, 0.0)


def _layer2_tc(acc_ref, cnt0_ref, cnt1_ref, x_ref, wl_ref, wr_ref, bl_ref,
               wa_ref, wb_ref, blin_ref, p_ref, q_ref):
    cnt = jnp.maximum(cnt0_ref[...] + cnt1_ref[...], 1.0)
    mean = (acc_ref[0] + acc_ref[1]) / cnt
    h = (lax.dot_general(mean, wl_ref[...], (((1,), (1,)), ((), ())),
                         preferred_element_type=jnp.float32)
         + lax.dot_general(x_ref[...], wr_ref[...], (((1,), (1,)), ((), ())),
                           preferred_element_type=jnp.float32)
         + bl_ref[...])
    p_ref[...] = jnp.dot(h, wa_ref[...],
                         preferred_element_type=jnp.float32) + blin_ref[...]
    q_ref[...] = jnp.dot(h, wb_ref[...], preferred_element_type=jnp.float32)


_BR = 1000


def _score_body(p_hbm, q_hbm, s_hbm, d_hbm, out_hbm, p_v, q_v, si_v, di_v, o_v):
    cid = lax.axis_index("c")
    sid = lax.axis_index("s")
    wid = cid * NS + sid
    base = pl.multiple_of(wid * LPW, 8)
    pltpu.sync_copy(p_hbm, p_v)
    pltpu.sync_copy(q_hbm, q_v)
    pltpu.sync_copy(s_hbm.at[pl.ds(base, LPW)], si_v)
    pltpu.sync_copy(d_hbm.at[pl.ds(base, LPW)], di_v)

    def step(j, _):
        sv = si_v[pl.ds(j * 16, 16)]
        dv = di_v[pl.ds(j * 16, 16)]
        o_v[pl.ds(j * 16, 16)] = (plsc.load_gather(p_v, [sv])
                                  + plsc.load_gather(q_v, [dv]))
        return 0
    lax.fori_loop(0, LPW // 16, step, 0)
    pltpu.sync_copy(o_v, out_hbm.at[pl.ds(base, LPW)])


_score = pl.kernel(
    _score_body,
    out_type=jax.ShapeDtypeStruct((NL,), jnp.float32),
    mesh=_mesh,
    compiler_params=pltpu.CompilerParams(needs_layout_passes=False),
    scratch_types=[
        pltpu.VMEM((N,), jnp.float32),
        pltpu.VMEM((N,), jnp.float32),
        pltpu.VMEM((LPW,), jnp.int32),
        pltpu.VMEM((LPW,), jnp.int32),
        pltpu.VMEM((LPW,), jnp.float32),
    ],
)


def _tc_layer(kind, acc, cnt0, cnt1, x, Wl, bl, Wr, wa=None, wb=None, blin=None):
    grid = (N // _BR,)
    acc_spec = pl.BlockSpec((NC, _BR, D), lambda i: (0, i, 0))
    cnt_spec = pl.BlockSpec((_BR, 1), lambda i: (i, 0))
    row_spec = pl.BlockSpec((_BR, D), lambda i: (i, 0))
    w_spec = pl.BlockSpec((D, D), lambda i: (0, 0))
    b_spec = pl.BlockSpec((1, D), lambda i: (0, 0))
    if kind == 1:
        return pl.pallas_call(
            _layer1_tc,
            grid=grid,
            in_specs=[acc_spec, cnt_spec, cnt_spec, row_spec, w_spec, w_spec,
                      b_spec],
            out_specs=row_spec,
            out_shape=jax.ShapeDtypeStruct((N, D), jnp.float32),
        )(acc, cnt0, cnt1, x, Wl, Wr, bl.reshape(1, D))
    v_spec = pl.BlockSpec((D, 1), lambda i: (0, 0))
    s_spec = pl.BlockSpec((1, 1), lambda i: (0, 0))
    pq_spec = pl.BlockSpec((_BR, 1), lambda i: (i, 0))
    return pl.pallas_call(
        _layer2_tc,
        grid=grid,
        in_specs=[acc_spec, cnt_spec, cnt_spec, row_spec, w_spec, w_spec,
                  b_spec, v_spec, v_spec, s_spec],
        out_specs=(pq_spec, pq_spec),
        out_shape=(jax.ShapeDtypeStruct((N, 1), jnp.float32),
                   jax.ShapeDtypeStruct((N, 1), jnp.float32)),
    )(acc, cnt0, cnt1, x, Wl, Wr, bl.reshape(1, D), wa, wb, blin.reshape(1, 1))


def kernel(edge_index, edge_label_index, emb, Wl1, bl1, Wr1, Wl2, bl2, Wr2,
           Wlin, blin):
    npad = E_PAD - E
    pad_src = (jnp.arange(npad, dtype=jnp.int32) * 37) % N
    pad_dst = N + (jnp.arange(npad, dtype=jnp.int32) % (ACC_PAD - N))
    src = jnp.concatenate([edge_index[0], pad_src]).reshape(NW, NCHUNK, CH)
    dst = jnp.concatenate([edge_index[1], pad_dst]).reshape(NW, NCHUNK, CH)

    acc1, cnt = _agg_with_counts(emb, src, dst)
    cnt0 = cnt[0, :N].reshape(N, 1)
    cnt1 = cnt[1, :N].reshape(N, 1)
    h1 = _tc_layer(1, acc1, cnt0, cnt1, emb, Wl1, bl1, Wr1)

    (acc2,) = _agg_no_counts(h1, src, dst)
    wa = Wlin[0, :D].reshape(D, 1)
    wb = Wlin[0, D:].reshape(D, 1)
    p, q = _tc_layer(2, acc2, cnt0, cnt1, h1, Wl2, bl2, Wr2, wa, wb, blin)

    out = _score(p.reshape(N), q.reshape(N),
                 edge_label_index[0], edge_label_index[1])
    return out


# speedup vs baseline: 1.3081x; 1.3081x over previous
import functools

import jax
import jax.numpy as jnp
from jax import lax
from jax.experimental import pallas as pl
from jax.experimental.pallas import tpu as pltpu
from jax.experimental.pallas import tpu_sc as plsc

N = 10000
D = 128
E = 320000
NL = 65536
NC, NS = 2, 16
NW = NC * NS
CH = 128
NCHUNK = 80
E_PAD = NW * NCHUNK * CH
NSTG = 2
CPS = NCHUNK // NSTG
PPS = CPS // 2
ACC_PAD = 10240
RPT = ACC_PAD // NS
ZR = 128
CNT_PAD = 10240
CPT = CNT_PAD // NS
LPW = NL // NW
ZW = 16

_mesh = plsc.VectorSubcoreMesh(core_axis_name="c", subcore_axis_name="s")


def _zero_f32(ref, nwords):
    def body(i, _):
        ref[pl.ds(i * 16, 16)] = jnp.zeros((16,), jnp.float32)
        return 0
    lax.fori_loop(0, nwords // 16, body, 0)


def _agg_body(with_counts, x_hbm, src_hbm, dst_hbm, acc_out, cnt_out,
              idx_s, idx_d, rows_a, rows_b, ones_v, zcnt_v,
              acc_sh, cnt_sh, gsa, gsb, ssa, ssb, csa, csb):
    cid = lax.axis_index("c")
    sid = lax.axis_index("s")
    gwid = cid * NS + sid

    def zrow_fill(i, _):
        rows_a[i // (D // 16), pl.ds((i % (D // 16)) * 16, 16)] = (
            jnp.zeros((16,), jnp.float32))
        return 0
    lax.fori_loop(0, ZR * (D // 16), zrow_fill, 0)

    def zacc(j, _):
        pltpu.sync_copy(rows_a, acc_sh.at[pl.ds(sid * RPT + j * ZR, ZR)])
        return 0
    lax.fori_loop(0, RPT // ZR, zacc, 0)

    if with_counts:
        def ones_fill(i, _):
            ones_v[pl.ds(i * 16, 16)] = jnp.ones((16,), jnp.float32)
            return 0
        lax.fori_loop(0, CH // 16, ones_fill, 0)
        _zero_f32(zcnt_v, CPT)
        pltpu.sync_copy(zcnt_v, cnt_sh.at[pl.ds(sid * CPT, CPT)])

    plsc.subcore_barrier()

    for st in range(NSTG):
        pltpu.sync_copy(src_hbm.at[gwid, pl.ds(st * CPS, CPS)], idx_s)
        pltpu.sync_copy(dst_hbm.at[gwid, pl.ds(st * CPS, CPS)], idx_d)
        pltpu.async_copy(x_hbm.at[idx_s.at[0]], rows_a, gsa)

        def pair(k2, _):
            ka = 2 * k2
            kb = ka + 1
            gb = pltpu.async_copy(x_hbm.at[idx_s.at[kb]], rows_b, gsb)
            pltpu.make_async_copy(x_hbm.at[idx_s.at[ka]], rows_a, gsa).wait()
            sa = pltpu.async_copy(rows_a, acc_sh.at[idx_d.at[ka]], ssa,
                                  add=True)
            if with_counts:
                ca = pltpu.async_copy(ones_v, cnt_sh.at[idx_d.at[ka]], csa,
                                      add=True)
            gb.wait()
            sb = pltpu.async_copy(rows_b, acc_sh.at[idx_d.at[kb]], ssb,
                                  add=True)
            if with_counts:
                cb = pltpu.async_copy(ones_v, cnt_sh.at[idx_d.at[kb]], csb,
                                      add=True)
            sa.wait()
            if with_counts:
                ca.wait()

            @pl.when(k2 < PPS - 1)
            def _():
                pltpu.async_copy(x_hbm.at[idx_s.at[ka + 2]], rows_a, gsa)
            sb.wait()
            if with_counts:
                cb.wait()
            return 0
        lax.fori_loop(0, PPS, pair, 0)

    plsc.subcore_barrier()

    def cout(j, _):
        r = pl.multiple_of(sid * RPT + j * ZR, 8)
        pltpu.sync_copy(acc_sh.at[pl.ds(r, ZR)], acc_out.at[cid, pl.ds(r, ZR)])
        return 0
    lax.fori_loop(0, RPT // ZR, cout, 0)
    if with_counts:
        pltpu.sync_copy(cnt_sh.at[pl.ds(sid * CPT, CPT)],
                        cnt_out.at[cid, pl.ds(sid * CPT, CPT)])


def _make_agg(with_counts):
    out_type = (jax.ShapeDtypeStruct((NC, ACC_PAD, D), jnp.float32),)
    if with_counts:
        out_type = out_type + (jax.ShapeDtypeStruct((NC, CNT_PAD), jnp.float32),)
    scratch = [
        pltpu.VMEM((CPS, CH), jnp.int32),
        pltpu.VMEM((CPS, CH), jnp.int32),
        pltpu.VMEM((CH, D), jnp.float32),
        pltpu.VMEM((CH, D), jnp.float32),
        pltpu.VMEM((CH,), jnp.float32),
        pltpu.VMEM((CPT,), jnp.float32),
        pltpu.VMEM_SHARED((ACC_PAD, D), jnp.float32),
        pltpu.VMEM_SHARED((CNT_PAD,), jnp.float32),
        pltpu.SemaphoreType.DMA,
        pltpu.SemaphoreType.DMA,
        pltpu.SemaphoreType.DMA,
        pltpu.SemaphoreType.DMA,
        pltpu.SemaphoreType.DMA,
        pltpu.SemaphoreType.DMA,
    ]
    body = functools.partial(_agg_body, with_counts)
    return pl.kernel(body, out_type=out_type, mesh=_mesh, scratch_types=scratch)


_agg_with_counts = _make_agg(True)


def _aggs_body(z_hbm, src_hbm, dst_hbm, acc_out,
               idx_s, idx_d, rows_a, rows_b, acc_sh, gsa, gsb, ssa, ssb):
    cid = lax.axis_index("c")
    sid = lax.axis_index("s")
    gwid = cid * NS + sid

    pltpu.sync_copy(src_hbm.at[gwid], idx_s)
    pltpu.sync_copy(dst_hbm.at[gwid], idx_d)

    def zrow_fill(i, _):
        rows_a[i, :] = jnp.zeros((ZW,), jnp.float32)
        return 0
    lax.fori_loop(0, CH, zrow_fill, 0)

    def zacc(j, _):
        pltpu.sync_copy(rows_a, acc_sh.at[pl.ds(sid * RPT + j * CH, CH)])
        return 0
    lax.fori_loop(0, RPT // CH, zacc, 0)
    plsc.subcore_barrier()

    pltpu.async_copy(z_hbm.at[idx_s.at[0]], rows_a, gsa)

    def pair(k2, _):
        ka = 2 * k2
        kb = ka + 1
        gb = pltpu.async_copy(z_hbm.at[idx_s.at[kb]], rows_b, gsb)
        pltpu.make_async_copy(z_hbm.at[idx_s.at[ka]], rows_a, gsa).wait()
        sa = pltpu.async_copy(rows_a, acc_sh.at[idx_d.at[ka]], ssa, add=True)
        gb.wait()
        sb = pltpu.async_copy(rows_b, acc_sh.at[idx_d.at[kb]], ssb, add=True)
        sa.wait()

        @pl.when(k2 < NCHUNK // 2 - 1)
        def _():
            pltpu.async_copy(z_hbm.at[idx_s.at[ka + 2]], rows_a, gsa)
        sb.wait()
        return 0
    lax.fori_loop(0, NCHUNK // 2, pair, 0)

    plsc.subcore_barrier()
    pltpu.sync_copy(acc_sh.at[pl.ds(sid * RPT, RPT)],
                    acc_out.at[cid, pl.ds(sid * RPT, RPT)])


_agg_scalar = pl.kernel(
    _aggs_body,
    out_type=jax.ShapeDtypeStruct((NC, ACC_PAD, ZW), jnp.float32),
    mesh=_mesh,
    compiler_params=pltpu.CompilerParams(use_tc_tiling_on_sc=False),
    scratch_types=[
        pltpu.VMEM((NCHUNK, CH), jnp.int32),
        pltpu.VMEM((NCHUNK, CH), jnp.int32),
        pltpu.VMEM((CH, ZW), jnp.float32),
        pltpu.VMEM((CH, ZW), jnp.float32),
        pltpu.VMEM_SHARED((ACC_PAD, ZW), jnp.float32),
        pltpu.SemaphoreType.DMA,
        pltpu.SemaphoreType.DMA,
        pltpu.SemaphoreType.DMA,
        pltpu.SemaphoreType.DMA,
    ],
)


def _layer1_tc(acc_ref, cnt0_ref, cnt1_ref, x_ref, wl_ref, wr_ref, bl_ref,
               wl2_ref, wab_ref, h_ref, z_ref):
    cnt = jnp.maximum(cnt0_ref[...] + cnt1_ref[...], 1.0)
    mean = (acc_ref[0] + acc_ref[1]) / cnt
    h = (lax.dot_general(mean, wl_ref[...], (((1,), (1,)), ((), ())),
                         preferred_element_type=jnp.float32)
         + lax.dot_general(x_ref[...], wr_ref[...], (((1,), (1,)), ((), ())),
                           preferred_element_type=jnp.float32)
         + bl_ref[...])
    h = jnp.maximum(h, 0.0)
    h_ref[...] = h
    w2 = lax.dot_general(wl2_ref[...], wab_ref[...], (((0,), (0,)), ((), ())),
                         preferred_element_type=jnp.float32)
    z_ref[...] = jnp.dot(h, w2, preferred_element_type=jnp.float32)


def _layer2_tc(zacc_ref, cnt0_ref, cnt1_ref, h_ref, wr2_ref, wab_ref,
               bl2_ref, blin_ref, p_ref, q_ref):
    cnt = jnp.maximum(cnt0_ref[...] + cnt1_ref[...], 1.0)
    mean_z = (zacc_ref[0] + zacc_ref[1]) / cnt
    u = lax.dot_general(wr2_ref[...], wab_ref[...], (((0,), (0,)), ((), ())),
                        preferred_element_type=jnp.float32)
    t = jnp.dot(h_ref[...], u, preferred_element_type=jnp.float32)
    cvec = lax.dot_general(bl2_ref[...], wab_ref[...],
                           (((1,), (0,)), ((), ())),
                           preferred_element_type=jnp.float32)
    s = mean_z + t + cvec
    p_ref[...] = s[:, 0:1] + blin_ref[...]
    q_ref[...] = s[:, 1:2]


_BR = 1000


def _score_body(p_hbm, q_hbm, s_hbm, d_hbm, out_hbm, p_v, q_v, si_v, di_v, o_v):
    cid = lax.axis_index("c")
    sid = lax.axis_index("s")
    wid = cid * NS + sid
    base = pl.multiple_of(wid * LPW, 8)
    pltpu.sync_copy(p_hbm, p_v)
    pltpu.sync_copy(q_hbm, q_v)
    pltpu.sync_copy(s_hbm.at[pl.ds(base, LPW)], si_v)
    pltpu.sync_copy(d_hbm.at[pl.ds(base, LPW)], di_v)

    def step(j, _):
        sv = si_v[pl.ds(j * 16, 16)]
        dv = di_v[pl.ds(j * 16, 16)]
        o_v[pl.ds(j * 16, 16)] = (plsc.load_gather(p_v, [sv])
                                  + plsc.load_gather(q_v, [dv]))
        return 0
    lax.fori_loop(0, LPW // 16, step, 0)
    pltpu.sync_copy(o_v, out_hbm.at[pl.ds(base, LPW)])


_score = pl.kernel(
    _score_body,
    out_type=jax.ShapeDtypeStruct((NL,), jnp.float32),
    mesh=_mesh,
    compiler_params=pltpu.CompilerParams(needs_layout_passes=False),
    scratch_types=[
        pltpu.VMEM((N,), jnp.float32),
        pltpu.VMEM((N,), jnp.float32),
        pltpu.VMEM((LPW,), jnp.int32),
        pltpu.VMEM((LPW,), jnp.int32),
        pltpu.VMEM((LPW,), jnp.float32),
    ],
)


_cnt_spec = pl.BlockSpec((_BR, 1), lambda i: (i, 0))
_row_spec = pl.BlockSpec((_BR, D), lambda i: (i, 0))
_w_spec = pl.BlockSpec((D, D), lambda i: (0, 0))
_b_spec = pl.BlockSpec((1, D), lambda i: (0, 0))
_wab_spec = pl.BlockSpec((D, ZW), lambda i: (0, 0))
_z_spec = pl.BlockSpec((_BR, ZW), lambda i: (i, 0))
_pq_spec = pl.BlockSpec((_BR, 1), lambda i: (i, 0))


def _tc_layer1(acc, cnt0, cnt1, x, Wl, Wr, bl, Wl2, wab):
    return pl.pallas_call(
        _layer1_tc,
        grid=(N // _BR,),
        in_specs=[pl.BlockSpec((NC, _BR, D), lambda i: (0, i, 0)),
                  _cnt_spec, _cnt_spec, _row_spec, _w_spec, _w_spec,
                  _b_spec, _w_spec, _wab_spec],
        out_specs=(_row_spec, _z_spec),
        out_shape=(jax.ShapeDtypeStruct((N, D), jnp.float32),
                   jax.ShapeDtypeStruct((N, ZW), jnp.float32)),
    )(acc, cnt0, cnt1, x, Wl, Wr, bl.reshape(1, D), Wl2, wab)


def _tc_layer2(zacc, cnt0, cnt1, h1, Wr2, wab, bl2, blin):
    return pl.pallas_call(
        _layer2_tc,
        grid=(N // _BR,),
        in_specs=[pl.BlockSpec((NC, _BR, ZW), lambda i: (0, i, 0)),
                  _cnt_spec, _cnt_spec, _row_spec, _w_spec, _wab_spec,
                  _b_spec, pl.BlockSpec((1, 1), lambda i: (0, 0))],
        out_specs=(_pq_spec, _pq_spec),
        out_shape=(jax.ShapeDtypeStruct((N, 1), jnp.float32),
                   jax.ShapeDtypeStruct((N, 1), jnp.float32)),
    )(zacc, cnt0, cnt1, h1, Wr2, wab, bl2.reshape(1, D), blin.reshape(1, 1))


def kernel(edge_index, edge_label_index, emb, Wl1, bl1, Wr1, Wl2, bl2, Wr2,
           Wlin, blin):
    npad = E_PAD - E
    pad_src = (jnp.arange(npad, dtype=jnp.int32) * 37) % N
    pad_dst = N + (jnp.arange(npad, dtype=jnp.int32) % (ACC_PAD - N))
    src = jnp.concatenate([edge_index[0], pad_src]).reshape(NW, NCHUNK, CH)
    dst = jnp.concatenate([edge_index[1], pad_dst]).reshape(NW, NCHUNK, CH)
    wab = jnp.zeros((D, ZW), jnp.float32)
    wab = wab.at[:, 0].set(Wlin[0, :D]).at[:, 1].set(Wlin[0, D:])

    acc1, cnt = _agg_with_counts(emb, src, dst)
    cnt0 = cnt[0, :N].reshape(N, 1)
    cnt1 = cnt[1, :N].reshape(N, 1)
    h1, z = _tc_layer1(acc1, cnt0, cnt1, emb, Wl1, Wr1, bl1, Wl2, wab)

    zacc = _agg_scalar(z, src, dst)
    p, q = _tc_layer2(zacc, cnt0, cnt1, h1, Wr2, wab, bl2, blin)

    out = _score(p.reshape(N), q.reshape(N),
                 edge_label_index[0], edge_label_index[1])
    return out


# speedup vs baseline: 1.3844x; 1.0584x over previous
import functools

import jax
import jax.numpy as jnp
from jax import lax
from jax.experimental import pallas as pl
from jax.experimental.pallas import tpu as pltpu
from jax.experimental.pallas import tpu_sc as plsc

N = 10000
D = 128
E = 320000
NL = 65536
NC, NS = 2, 16
NW = NC * NS
CH = 128
NCHUNK = 80
E_PAD = NW * NCHUNK * CH
NSTG = 2
CPS = NCHUNK // NSTG
PPS = CPS // 2
ACC_PAD = 10240
RPT = ACC_PAD // NS
ZR = 128
CNT_PAD = 10240
CPT = CNT_PAD // NS
LPW = NL // NW
ZW = 16

_mesh = plsc.VectorSubcoreMesh(core_axis_name="c", subcore_axis_name="s")


def _zero_f32(ref, nwords):
    def body(i, _):
        ref[pl.ds(i * 16, 16)] = jnp.zeros((16,), jnp.float32)
        return 0
    lax.fori_loop(0, nwords // 16, body, 0)


def _agg_body(with_counts, x_hbm, src_hbm, dst_hbm, acc_out, cnt_out,
              idx_s, idx_d, rows_a, rows_b, ones_v, zcnt_v,
              acc_sh, cnt_sh, gsa, gsb, ssa, ssb, csa, csb):
    cid = lax.axis_index("c")
    sid = lax.axis_index("s")
    gwid = cid * NS + sid

    def zrow_fill(i, _):
        rows_a[i // (D // 16), pl.ds((i % (D // 16)) * 16, 16)] = (
            jnp.zeros((16,), jnp.float32))
        return 0
    lax.fori_loop(0, ZR * (D // 16), zrow_fill, 0)

    def zacc(j, _):
        pltpu.sync_copy(rows_a, acc_sh.at[pl.ds(sid * RPT + j * ZR, ZR)])
        return 0
    lax.fori_loop(0, RPT // ZR, zacc, 0)

    if with_counts:
        def ones_fill(i, _):
            ones_v[pl.ds(i * 16, 16)] = jnp.ones((16,), jnp.float32)
            return 0
        lax.fori_loop(0, CH // 16, ones_fill, 0)
        _zero_f32(zcnt_v, CPT)
        pltpu.sync_copy(zcnt_v, cnt_sh.at[pl.ds(sid * CPT, CPT)])

    plsc.subcore_barrier()

    for st in range(NSTG):
        pltpu.sync_copy(src_hbm.at[gwid, pl.ds(st * CPS, CPS)], idx_s)
        pltpu.sync_copy(dst_hbm.at[gwid, pl.ds(st * CPS, CPS)], idx_d)
        pltpu.async_copy(x_hbm.at[idx_s.at[0]], rows_a, gsa)

        def pair(k2, _):
            ka = 2 * k2
            kb = ka + 1
            pltpu.make_async_copy(x_hbm.at[idx_s.at[ka]], rows_a, gsa).wait()
            gb = pltpu.async_copy(x_hbm.at[idx_s.at[kb]], rows_b, gsb)
            sa = pltpu.async_copy(rows_a, acc_sh.at[idx_d.at[ka]], ssa,
                                  add=True)
            if with_counts:
                ca = pltpu.async_copy(ones_v, cnt_sh.at[idx_d.at[ka]], csa,
                                      add=True)
            gb.wait()
            sa.wait()
            if with_counts:
                ca.wait()

            @pl.when(k2 < PPS - 1)
            def _():
                pltpu.async_copy(x_hbm.at[idx_s.at[ka + 2]], rows_a, gsa)
            sb = pltpu.async_copy(rows_b, acc_sh.at[idx_d.at[kb]], ssb,
                                  add=True)
            if with_counts:
                cb = pltpu.async_copy(ones_v, cnt_sh.at[idx_d.at[kb]], csb,
                                      add=True)
            sb.wait()
            if with_counts:
                cb.wait()
            return 0
        lax.fori_loop(0, PPS, pair, 0)

    plsc.subcore_barrier()

    def cout(j, _):
        r = pl.multiple_of(sid * RPT + j * ZR, 8)
        pltpu.sync_copy(acc_sh.at[pl.ds(r, ZR)], acc_out.at[cid, pl.ds(r, ZR)])
        return 0
    lax.fori_loop(0, RPT // ZR, cout, 0)
    if with_counts:
        pltpu.sync_copy(cnt_sh.at[pl.ds(sid * CPT, CPT)],
                        cnt_out.at[cid, pl.ds(sid * CPT, CPT)])


def _make_agg(with_counts):
    out_type = (jax.ShapeDtypeStruct((NC, ACC_PAD, D), jnp.float32),)
    if with_counts:
        out_type = out_type + (jax.ShapeDtypeStruct((NC, CNT_PAD), jnp.float32),)
    scratch = [
        pltpu.VMEM((CPS, CH), jnp.int32),
        pltpu.VMEM((CPS, CH), jnp.int32),
        pltpu.VMEM((CH, D), jnp.float32),
        pltpu.VMEM((CH, D), jnp.float32),
        pltpu.VMEM((CH,), jnp.float32),
        pltpu.VMEM((CPT,), jnp.float32),
        pltpu.VMEM_SHARED((ACC_PAD, D), jnp.float32),
        pltpu.VMEM_SHARED((CNT_PAD,), jnp.float32),
        pltpu.SemaphoreType.DMA,
        pltpu.SemaphoreType.DMA,
        pltpu.SemaphoreType.DMA,
        pltpu.SemaphoreType.DMA,
        pltpu.SemaphoreType.DMA,
        pltpu.SemaphoreType.DMA,
    ]
    body = functools.partial(_agg_body, with_counts)
    return pl.kernel(body, out_type=out_type, mesh=_mesh, scratch_types=scratch)


_agg_with_counts = _make_agg(True)


def _aggs_body(z_hbm, src_hbm, dst_hbm, acc_out,
               idx_s, idx_d, rows_a, rows_b, acc_sh, gsa, gsb, ssa, ssb):
    cid = lax.axis_index("c")
    sid = lax.axis_index("s")
    gwid = cid * NS + sid

    pltpu.sync_copy(src_hbm.at[gwid], idx_s)
    pltpu.sync_copy(dst_hbm.at[gwid], idx_d)

    def zrow_fill(i, _):
        rows_a[i, :] = jnp.zeros((ZW,), jnp.float32)
        return 0
    lax.fori_loop(0, CH, zrow_fill, 0)

    def zacc(j, _):
        pltpu.sync_copy(rows_a, acc_sh.at[pl.ds(sid * RPT + j * CH, CH)])
        return 0
    lax.fori_loop(0, RPT // CH, zacc, 0)
    plsc.subcore_barrier()

    pltpu.async_copy(z_hbm.at[idx_s.at[0]], rows_a, gsa)

    def pair(k2, _):
        ka = 2 * k2
        kb = ka + 1
        gb = pltpu.async_copy(z_hbm.at[idx_s.at[kb]], rows_b, gsb)
        pltpu.make_async_copy(z_hbm.at[idx_s.at[ka]], rows_a, gsa).wait()
        sa = pltpu.async_copy(rows_a, acc_sh.at[idx_d.at[ka]], ssa, add=True)
        gb.wait()
        sb = pltpu.async_copy(rows_b, acc_sh.at[idx_d.at[kb]], ssb, add=True)
        sa.wait()

        @pl.when(k2 < NCHUNK // 2 - 1)
        def _():
            pltpu.async_copy(z_hbm.at[idx_s.at[ka + 2]], rows_a, gsa)
        sb.wait()
        return 0
    lax.fori_loop(0, NCHUNK // 2, pair, 0)

    plsc.subcore_barrier()
    pltpu.sync_copy(acc_sh.at[pl.ds(sid * RPT, RPT)],
                    acc_out.at[cid, pl.ds(sid * RPT, RPT)])


_agg_scalar = pl.kernel(
    _aggs_body,
    out_type=jax.ShapeDtypeStruct((NC, ACC_PAD, ZW), jnp.float32),
    mesh=_mesh,
    compiler_params=pltpu.CompilerParams(use_tc_tiling_on_sc=False),
    scratch_types=[
        pltpu.VMEM((NCHUNK, CH), jnp.int32),
        pltpu.VMEM((NCHUNK, CH), jnp.int32),
        pltpu.VMEM((CH, ZW), jnp.float32),
        pltpu.VMEM((CH, ZW), jnp.float32),
        pltpu.VMEM_SHARED((ACC_PAD, ZW), jnp.float32),
        pltpu.SemaphoreType.DMA,
        pltpu.SemaphoreType.DMA,
        pltpu.SemaphoreType.DMA,
        pltpu.SemaphoreType.DMA,
    ],
)


def _layer1_tc(acc_ref, cnt0_ref, cnt1_ref, x_ref, wl_ref, wr_ref, bl_ref,
               wl2_ref, wab_ref, h_ref, z_ref):
    cnt = jnp.maximum(cnt0_ref[...] + cnt1_ref[...], 1.0)
    mean = (acc_ref[0] + acc_ref[1]) / cnt
    h = (lax.dot_general(mean, wl_ref[...], (((1,), (1,)), ((), ())),
                         preferred_element_type=jnp.float32)
         + lax.dot_general(x_ref[...], wr_ref[...], (((1,), (1,)), ((), ())),
                           preferred_element_type=jnp.float32)
         + bl_ref[...])
    h = jnp.maximum(h, 0.0)
    h_ref[...] = h
    w2 = lax.dot_general(wl2_ref[...], wab_ref[...], (((0,), (0,)), ((), ())),
                         preferred_element_type=jnp.float32)
    z_ref[...] = jnp.dot(h, w2, preferred_element_type=jnp.float32)


def _layer2_tc(zacc_ref, cnt0_ref, cnt1_ref, h_ref, wr2_ref, wab_ref,
               bl2_ref, blin_ref, p_ref, q_ref):
    cnt = jnp.maximum(cnt0_ref[...] + cnt1_ref[...], 1.0)
    mean_z = (zacc_ref[0] + zacc_ref[1]) / cnt
    u = lax.dot_general(wr2_ref[...], wab_ref[...], (((0,), (0,)), ((), ())),
                        preferred_element_type=jnp.float32)
    t = jnp.dot(h_ref[...], u, preferred_element_type=jnp.float32)
    cvec = lax.dot_general(bl2_ref[...], wab_ref[...],
                           (((1,), (0,)), ((), ())),
                           preferred_element_type=jnp.float32)
    s = mean_z + t + cvec
    p_ref[...] = s[:, 0:1] + blin_ref[...]
    q_ref[...] = s[:, 1:2]


_BR = 1000


def _score_body(p_hbm, q_hbm, s_hbm, d_hbm, out_hbm, p_v, q_v, si_v, di_v, o_v):
    cid = lax.axis_index("c")
    sid = lax.axis_index("s")
    wid = cid * NS + sid
    base = pl.multiple_of(wid * LPW, 8)
    pltpu.sync_copy(p_hbm, p_v)
    pltpu.sync_copy(q_hbm, q_v)
    pltpu.sync_copy(s_hbm.at[pl.ds(base, LPW)], si_v)
    pltpu.sync_copy(d_hbm.at[pl.ds(base, LPW)], di_v)

    def step(j, _):
        sv = si_v[pl.ds(j * 16, 16)]
        dv = di_v[pl.ds(j * 16, 16)]
        o_v[pl.ds(j * 16, 16)] = (plsc.load_gather(p_v, [sv])
                                  + plsc.load_gather(q_v, [dv]))
        return 0
    lax.fori_loop(0, LPW // 16, step, 0)
    pltpu.sync_copy(o_v, out_hbm.at[pl.ds(base, LPW)])


_score = pl.kernel(
    _score_body,
    out_type=jax.ShapeDtypeStruct((NL,), jnp.float32),
    mesh=_mesh,
    compiler_params=pltpu.CompilerParams(needs_layout_passes=False),
    scratch_types=[
        pltpu.VMEM((N,), jnp.float32),
        pltpu.VMEM((N,), jnp.float32),
        pltpu.VMEM((LPW,), jnp.int32),
        pltpu.VMEM((LPW,), jnp.int32),
        pltpu.VMEM((LPW,), jnp.float32),
    ],
)


_cnt_spec = pl.BlockSpec((_BR, 1), lambda i: (i, 0))
_row_spec = pl.BlockSpec((_BR, D), lambda i: (i, 0))
_w_spec = pl.BlockSpec((D, D), lambda i: (0, 0))
_b_spec = pl.BlockSpec((1, D), lambda i: (0, 0))
_wab_spec = pl.BlockSpec((D, ZW), lambda i: (0, 0))
_z_spec = pl.BlockSpec((_BR, ZW), lambda i: (i, 0))
_pq_spec = pl.BlockSpec((_BR, 1), lambda i: (i, 0))


def _tc_layer1(acc, cnt0, cnt1, x, Wl, Wr, bl, Wl2, wab):
    return pl.pallas_call(
        _layer1_tc,
        grid=(N // _BR,),
        in_specs=[pl.BlockSpec((NC, _BR, D), lambda i: (0, i, 0)),
                  _cnt_spec, _cnt_spec, _row_spec, _w_spec, _w_spec,
                  _b_spec, _w_spec, _wab_spec],
        out_specs=(_row_spec, _z_spec),
        out_shape=(jax.ShapeDtypeStruct((N, D), jnp.float32),
                   jax.ShapeDtypeStruct((N, ZW), jnp.float32)),
    )(acc, cnt0, cnt1, x, Wl, Wr, bl.reshape(1, D), Wl2, wab)


def _tc_layer2(zacc, cnt0, cnt1, h1, Wr2, wab, bl2, blin):
    return pl.pallas_call(
        _layer2_tc,
        grid=(N // _BR,),
        in_specs=[pl.BlockSpec((NC, _BR, ZW), lambda i: (0, i, 0)),
                  _cnt_spec, _cnt_spec, _row_spec, _w_spec, _wab_spec,
                  _b_spec, pl.BlockSpec((1, 1), lambda i: (0, 0))],
        out_specs=(_pq_spec, _pq_spec),
        out_shape=(jax.ShapeDtypeStruct((N, 1), jnp.float32),
                   jax.ShapeDtypeStruct((N, 1), jnp.float32)),
    )(zacc, cnt0, cnt1, h1, Wr2, wab, bl2.reshape(1, D), blin.reshape(1, 1))


def kernel(edge_index, edge_label_index, emb, Wl1, bl1, Wr1, Wl2, bl2, Wr2,
           Wlin, blin):
    npad = E_PAD - E
    pad_src = (jnp.arange(npad, dtype=jnp.int32) * 37) % N
    pad_dst = N + (jnp.arange(npad, dtype=jnp.int32) % (ACC_PAD - N))
    src = jnp.concatenate([edge_index[0], pad_src]).reshape(NW, NCHUNK, CH)
    dst = jnp.concatenate([edge_index[1], pad_dst]).reshape(NW, NCHUNK, CH)
    wab = jnp.zeros((D, ZW), jnp.float32)
    wab = wab.at[:, 0].set(Wlin[0, :D]).at[:, 1].set(Wlin[0, D:])

    acc1, cnt = _agg_with_counts(emb, src, dst)
    cnt0 = cnt[0, :N].reshape(N, 1)
    cnt1 = cnt[1, :N].reshape(N, 1)
    h1, z = _tc_layer1(acc1, cnt0, cnt1, emb, Wl1, Wr1, bl1, Wl2, wab)

    zacc = _agg_scalar(z, src, dst)
    p, q = _tc_layer2(zacc, cnt0, cnt1, h1, Wr2, wab, bl2, blin)

    out = _score(p.reshape(N), q.reshape(N),
                 edge_label_index[0], edge_label_index[1])
    return out


# speedup vs baseline: 1.5131x; 1.0930x over previous
import functools

import jax
import jax.numpy as jnp
from jax import lax
from jax.experimental import pallas as pl
from jax.experimental.pallas import tpu as pltpu
from jax.experimental.pallas import tpu_sc as plsc

N = 10000
D = 128
E = 320000
NL = 65536
NC, NS = 2, 16
NW = NC * NS
CH = 128
NCHUNK = 80
E_PAD = NW * NCHUNK * CH
NSTG = 2
CPS = NCHUNK // NSTG
PPS = CPS // 2
ACC_PAD = 10240
RPT = ACC_PAD // NS
ZR = 128
CNT_PAD = 10240
CPT = CNT_PAD // NS
LPW = NL // NW
ZW = 16

_mesh = plsc.VectorSubcoreMesh(core_axis_name="c", subcore_axis_name="s")


def _zero_f32(ref, nwords):
    def body(i, _):
        ref[pl.ds(i * 16, 16)] = jnp.zeros((16,), jnp.float32)
        return 0
    lax.fori_loop(0, nwords // 16, body, 0)


def _agg_body(with_counts, x_hbm, src_hbm, dst_hbm, acc_out, cnt_out,
              idx_s, idx_d, rows_a, rows_b, ones_v, zcnt_v,
              acc_sh, cnt_sh, gsa, gsb, ssa, ssb, csa, csb):
    cid = lax.axis_index("c")
    sid = lax.axis_index("s")
    gwid = cid * NS + sid

    def zrow_fill(i, _):
        rows_a[i // (D // 16), pl.ds((i % (D // 16)) * 16, 16)] = (
            jnp.zeros((16,), jnp.float32))
        return 0
    lax.fori_loop(0, ZR * (D // 16), zrow_fill, 0)

    def zacc(j, _):
        pltpu.sync_copy(rows_a, acc_sh.at[pl.ds(sid * RPT + j * ZR, ZR)])
        return 0
    lax.fori_loop(0, RPT // ZR, zacc, 0)

    if with_counts:
        def ones_fill(i, _):
            ones_v[pl.ds(i * 16, 16)] = jnp.ones((16,), jnp.float32)
            return 0
        lax.fori_loop(0, CH // 16, ones_fill, 0)
        _zero_f32(zcnt_v, CPT)
        pltpu.sync_copy(zcnt_v, cnt_sh.at[pl.ds(sid * CPT, CPT)])

    plsc.subcore_barrier()

    for st in range(NSTG):
        pltpu.sync_copy(src_hbm.at[gwid, pl.ds(st * CPS, CPS)], idx_s)
        pltpu.sync_copy(dst_hbm.at[gwid, pl.ds(st * CPS, CPS)], idx_d)
        pltpu.async_copy(x_hbm.at[idx_s.at[0]], rows_a, gsa)

        def pair(k2, _):
            ka = 2 * k2
            kb = ka + 1
            pltpu.make_async_copy(x_hbm.at[idx_s.at[ka]], rows_a, gsa).wait()
            gb = pltpu.async_copy(x_hbm.at[idx_s.at[kb]], rows_b, gsb)
            sa = pltpu.async_copy(rows_a, acc_sh.at[idx_d.at[ka]], ssa,
                                  add=True)
            if with_counts:
                ca = pltpu.async_copy(ones_v, cnt_sh.at[idx_d.at[ka]], csa,
                                      add=True)
            gb.wait()
            sa.wait()
            if with_counts:
                ca.wait()

            @pl.when(k2 < PPS - 1)
            def _():
                pltpu.async_copy(x_hbm.at[idx_s.at[ka + 2]], rows_a, gsa)
            sb = pltpu.async_copy(rows_b, acc_sh.at[idx_d.at[kb]], ssb,
                                  add=True)
            if with_counts:
                cb = pltpu.async_copy(ones_v, cnt_sh.at[idx_d.at[kb]], csb,
                                      add=True)
            sb.wait()
            if with_counts:
                cb.wait()
            return 0
        lax.fori_loop(0, PPS, pair, 0)

    plsc.subcore_barrier()

    def cout(j, _):
        r = pl.multiple_of(sid * RPT + j * ZR, 8)
        pltpu.sync_copy(acc_sh.at[pl.ds(r, ZR)], acc_out.at[cid, pl.ds(r, ZR)])
        return 0
    lax.fori_loop(0, RPT // ZR, cout, 0)
    if with_counts:
        pltpu.sync_copy(cnt_sh.at[pl.ds(sid * CPT, CPT)],
                        cnt_out.at[cid, pl.ds(sid * CPT, CPT)])


def _make_agg(with_counts):
    out_type = (jax.ShapeDtypeStruct((NC, ACC_PAD, D), jnp.float32),)
    if with_counts:
        out_type = out_type + (jax.ShapeDtypeStruct((NC, CNT_PAD), jnp.float32),)
    scratch = [
        pltpu.VMEM((CPS, CH), jnp.int32),
        pltpu.VMEM((CPS, CH), jnp.int32),
        pltpu.VMEM((CH, D), jnp.float32),
        pltpu.VMEM((CH, D), jnp.float32),
        pltpu.VMEM((CH,), jnp.float32),
        pltpu.VMEM((CPT,), jnp.float32),
        pltpu.VMEM_SHARED((ACC_PAD, D), jnp.float32),
        pltpu.VMEM_SHARED((CNT_PAD,), jnp.float32),
        pltpu.SemaphoreType.DMA,
        pltpu.SemaphoreType.DMA,
        pltpu.SemaphoreType.DMA,
        pltpu.SemaphoreType.DMA,
        pltpu.SemaphoreType.DMA,
        pltpu.SemaphoreType.DMA,
    ]
    body = functools.partial(_agg_body, with_counts)
    return pl.kernel(body, out_type=out_type, mesh=_mesh, scratch_types=scratch)


_agg_with_counts = _make_agg(True)


def _aggs_body(z_hbm, src_hbm, dst_hbm, acc_out,
               idx_s, idx_d, r0, r1, r2, r3,
               acc_sh, g0, g1, g2, g3, s0, s1, s2, s3):
    cid = lax.axis_index("c")
    sid = lax.axis_index("s")
    gwid = cid * NS + sid
    rows = (r0, r1, r2, r3)
    gs = (g0, g1, g2, g3)
    ss = (s0, s1, s2, s3)

    pltpu.sync_copy(src_hbm.at[gwid], idx_s)
    pltpu.sync_copy(dst_hbm.at[gwid], idx_d)

    def zrow_fill(i, _):
        r0[i, :] = jnp.zeros((ZW,), jnp.float32)
        return 0
    lax.fori_loop(0, CH, zrow_fill, 0)

    def zacc(j, _):
        pltpu.sync_copy(r0, acc_sh.at[pl.ds(sid * RPT + j * CH, CH)])
        return 0
    lax.fori_loop(0, RPT // CH, zacc, 0)
    plsc.subcore_barrier()

    for j in range(3):
        pltpu.async_copy(z_hbm.at[idx_s.at[j]], rows[j], gs[j])

    def quad(k4, _):
        for j in range(4):
            c = 4 * k4 + j
            pltpu.make_async_copy(z_hbm.at[idx_s.at[c]], rows[j],
                                  gs[j]).wait()
            pltpu.async_copy(rows[j], acc_sh.at[idx_d.at[c]], ss[j],
                             add=True)
            jp = (j + 3) % 4
            cp = c + 3

            @pl.when(cp < NCHUNK)
            def _():
                @pl.when(c > 0)
                def _():
                    pltpu.make_async_copy(
                        rows[jp], acc_sh.at[idx_d.at[0]], ss[jp]).wait()
                pltpu.async_copy(z_hbm.at[idx_s.at[cp]], rows[jp], gs[jp])
        return 0
    lax.fori_loop(0, NCHUNK // 4, quad, 0)
    for j in range(4):
        pltpu.make_async_copy(rows[j], acc_sh.at[idx_d.at[0]], ss[j]).wait()

    plsc.subcore_barrier()
    pltpu.sync_copy(acc_sh.at[pl.ds(sid * RPT, RPT)],
                    acc_out.at[cid, pl.ds(sid * RPT, RPT)])


_agg_scalar = pl.kernel(
    _aggs_body,
    out_type=jax.ShapeDtypeStruct((NC, ACC_PAD, ZW), jnp.float32),
    mesh=_mesh,
    compiler_params=pltpu.CompilerParams(use_tc_tiling_on_sc=False),
    scratch_types=[
        pltpu.VMEM((NCHUNK, CH), jnp.int32),
        pltpu.VMEM((NCHUNK, CH), jnp.int32),
        pltpu.VMEM((CH, ZW), jnp.float32),
        pltpu.VMEM((CH, ZW), jnp.float32),
        pltpu.VMEM((CH, ZW), jnp.float32),
        pltpu.VMEM((CH, ZW), jnp.float32),
        pltpu.VMEM_SHARED((ACC_PAD, ZW), jnp.float32),
        pltpu.SemaphoreType.DMA,
        pltpu.SemaphoreType.DMA,
        pltpu.SemaphoreType.DMA,
        pltpu.SemaphoreType.DMA,
        pltpu.SemaphoreType.DMA,
        pltpu.SemaphoreType.DMA,
        pltpu.SemaphoreType.DMA,
        pltpu.SemaphoreType.DMA,
    ],
)


def _layer1_tc(acc_ref, cnt0_ref, cnt1_ref, x_ref, wl_ref, wr_ref, bl_ref,
               wl2_ref, wab_ref, h_ref, z_ref):
    cnt = jnp.maximum(cnt0_ref[...] + cnt1_ref[...], 1.0)
    mean = (acc_ref[0] + acc_ref[1]) / cnt
    h = (lax.dot_general(mean, wl_ref[...], (((1,), (1,)), ((), ())),
                         preferred_element_type=jnp.float32)
         + lax.dot_general(x_ref[...], wr_ref[...], (((1,), (1,)), ((), ())),
                           preferred_element_type=jnp.float32)
         + bl_ref[...])
    h = jnp.maximum(h, 0.0)
    h_ref[...] = h
    w2 = lax.dot_general(wl2_ref[...], wab_ref[...], (((0,), (0,)), ((), ())),
                         preferred_element_type=jnp.float32)
    z_ref[...] = jnp.dot(h, w2, preferred_element_type=jnp.float32)


def _layer2_tc(zacc_ref, cnt0_ref, cnt1_ref, h_ref, wr2_ref, wab_ref,
               bl2_ref, blin_ref, p_ref, q_ref):
    cnt = jnp.maximum(cnt0_ref[...] + cnt1_ref[...], 1.0)
    mean_z = (zacc_ref[0] + zacc_ref[1]) / cnt
    u = lax.dot_general(wr2_ref[...], wab_ref[...], (((0,), (0,)), ((), ())),
                        preferred_element_type=jnp.float32)
    t = jnp.dot(h_ref[...], u, preferred_element_type=jnp.float32)
    cvec = lax.dot_general(bl2_ref[...], wab_ref[...],
                           (((1,), (0,)), ((), ())),
                           preferred_element_type=jnp.float32)
    s = mean_z + t + cvec
    p_ref[...] = s[:, 0:1] + blin_ref[...]
    q_ref[...] = s[:, 1:2]


_BR = 1000


def _score_body(p_hbm, q_hbm, s_hbm, d_hbm, out_hbm, p_v, q_v, si_v, di_v, o_v):
    cid = lax.axis_index("c")
    sid = lax.axis_index("s")
    wid = cid * NS + sid
    base = pl.multiple_of(wid * LPW, 8)
    pltpu.sync_copy(p_hbm, p_v)
    pltpu.sync_copy(q_hbm, q_v)
    pltpu.sync_copy(s_hbm.at[pl.ds(base, LPW)], si_v)
    pltpu.sync_copy(d_hbm.at[pl.ds(base, LPW)], di_v)

    def step(j, _):
        sv = si_v[pl.ds(j * 16, 16)]
        dv = di_v[pl.ds(j * 16, 16)]
        o_v[pl.ds(j * 16, 16)] = (plsc.load_gather(p_v, [sv])
                                  + plsc.load_gather(q_v, [dv]))
        return 0
    lax.fori_loop(0, LPW // 16, step, 0)
    pltpu.sync_copy(o_v, out_hbm.at[pl.ds(base, LPW)])


_score = pl.kernel(
    _score_body,
    out_type=jax.ShapeDtypeStruct((NL,), jnp.float32),
    mesh=_mesh,
    compiler_params=pltpu.CompilerParams(needs_layout_passes=False),
    scratch_types=[
        pltpu.VMEM((N,), jnp.float32),
        pltpu.VMEM((N,), jnp.float32),
        pltpu.VMEM((LPW,), jnp.int32),
        pltpu.VMEM((LPW,), jnp.int32),
        pltpu.VMEM((LPW,), jnp.float32),
    ],
)


_cnt_spec = pl.BlockSpec((_BR, 1), lambda i: (i, 0))
_row_spec = pl.BlockSpec((_BR, D), lambda i: (i, 0))
_w_spec = pl.BlockSpec((D, D), lambda i: (0, 0))
_b_spec = pl.BlockSpec((1, D), lambda i: (0, 0))
_wab_spec = pl.BlockSpec((D, ZW), lambda i: (0, 0))
_z_spec = pl.BlockSpec((_BR, ZW), lambda i: (i, 0))
_pq_spec = pl.BlockSpec((_BR, 1), lambda i: (i, 0))


def _tc_layer1(acc, cnt0, cnt1, x, Wl, Wr, bl, Wl2, wab):
    return pl.pallas_call(
        _layer1_tc,
        grid=(N // _BR,),
        in_specs=[pl.BlockSpec((NC, _BR, D), lambda i: (0, i, 0)),
                  _cnt_spec, _cnt_spec, _row_spec, _w_spec, _w_spec,
                  _b_spec, _w_spec, _wab_spec],
        out_specs=(_row_spec, _z_spec),
        out_shape=(jax.ShapeDtypeStruct((N, D), jnp.float32),
                   jax.ShapeDtypeStruct((N, ZW), jnp.float32)),
    )(acc, cnt0, cnt1, x, Wl, Wr, bl.reshape(1, D), Wl2, wab)


def _tc_layer2(zacc, cnt0, cnt1, h1, Wr2, wab, bl2, blin):
    return pl.pallas_call(
        _layer2_tc,
        grid=(N // _BR,),
        in_specs=[pl.BlockSpec((NC, _BR, ZW), lambda i: (0, i, 0)),
                  _cnt_spec, _cnt_spec, _row_spec, _w_spec, _wab_spec,
                  _b_spec, pl.BlockSpec((1, 1), lambda i: (0, 0))],
        out_specs=(_pq_spec, _pq_spec),
        out_shape=(jax.ShapeDtypeStruct((N, 1), jnp.float32),
                   jax.ShapeDtypeStruct((N, 1), jnp.float32)),
    )(zacc, cnt0, cnt1, h1, Wr2, wab, bl2.reshape(1, D), blin.reshape(1, 1))


def kernel(edge_index, edge_label_index, emb, Wl1, bl1, Wr1, Wl2, bl2, Wr2,
           Wlin, blin):
    npad = E_PAD - E
    pad_src = (jnp.arange(npad, dtype=jnp.int32) * 37) % N
    pad_dst = N + (jnp.arange(npad, dtype=jnp.int32) % (ACC_PAD - N))
    src = jnp.concatenate([edge_index[0], pad_src]).reshape(NW, NCHUNK, CH)
    dst = jnp.concatenate([edge_index[1], pad_dst]).reshape(NW, NCHUNK, CH)
    wab = jnp.zeros((D, ZW), jnp.float32)
    wab = wab.at[:, 0].set(Wlin[0, :D]).at[:, 1].set(Wlin[0, D:])

    acc1, cnt = _agg_with_counts(emb, src, dst)
    cnt0 = cnt[0, :N].reshape(N, 1)
    cnt1 = cnt[1, :N].reshape(N, 1)
    h1, z = _tc_layer1(acc1, cnt0, cnt1, emb, Wl1, Wr1, bl1, Wl2, wab)

    zacc = _agg_scalar(z, src, dst)
    p, q = _tc_layer2(zacc, cnt0, cnt1, h1, Wr2, wab, bl2, blin)

    out = _score(p.reshape(N), q.reshape(N),
                 edge_label_index[0], edge_label_index[1])
    return out
